# Initial kernel scaffold; baseline (speedup 1.0000x reference)
import jax
import jax.numpy as jnp
from jax.experimental import pallas as pl


def kernel(adj_indices, adj_values, embeds):
    raise NotImplementedError("write your pallas kernel here")


# speedup vs baseline: 4.5018x; 4.5018x over previous
import functools

import jax
import jax.numpy as jnp
from jax import lax
from jax.experimental import pallas as pl
from jax.experimental.pallas import tpu as pltpu
from jax.experimental.pallas import tpu_sc as plsc

N_NODES = 10000
N_EDGES = 320000
D_FEAT = 128

NC = 2
NS = 16
LANES = 16
NW = NC * NS
E_PER_W = N_EDGES // NW
BATCH = 80
NBATCH = E_PER_W // BATCH
N_PAD = 10240
ROWS_PER_TILE = N_PAD // NS
ZROWS = 128
NSEG = D_FEAT // LANES


def _sc_spmm(rows_hbm, cols_hbm, vals_hbm, embeds_hbm,
             out0, out1,
             gbuf, colv, rowv, valv, zbuf, acc, sem):
    c = lax.axis_index("c")
    s = lax.axis_index("s")
    wid = c * NS + s
    base = wid * E_PER_W

    zeros16 = jnp.zeros((LANES,), jnp.float32)

    def zero_body(i, _):
        for k in range(NSEG):
            zbuf[i, pl.ds(k * LANES, LANES)] = zeros16
        return 0

    lax.fori_loop(0, ZROWS, zero_body, 0)
    for j in range(ROWS_PER_TILE // ZROWS):
        pltpu.sync_copy(zbuf, acc.at[pl.ds(s * ROWS_PER_TILE + j * ZROWS, ZROWS)])
    plsc.subcore_barrier()

    def batch_body(b, _):
        off = base + b * BATCH
        pltpu.sync_copy(cols_hbm.at[pl.ds(off, BATCH)], colv)
        pltpu.sync_copy(rows_hbm.at[pl.ds(off, BATCH)], rowv)
        pltpu.sync_copy(vals_hbm.at[pl.ds(off, BATCH)], valv)
        pltpu.async_copy(embeds_hbm.at[colv], gbuf, sem).wait()

        def scale_body(j, _):
            v16 = valv[pl.ds(j * LANES, LANES)]
            ebase = j * LANES
            for i in range(LANES):
                vv = jnp.full((LANES,), v16[i], jnp.float32)
                for k in range(NSEG):
                    sl = pl.ds(k * LANES, LANES)
                    gbuf[ebase + i, sl] = gbuf[ebase + i, sl] * vv
            return 0

        lax.fori_loop(0, BATCH // LANES, scale_body, 0)
        pltpu.sync_copy(gbuf, acc.at[rowv], add=True)
        return 0

    lax.fori_loop(0, NBATCH, batch_body, 0)
    plsc.subcore_barrier()

    rsl = pl.ds(s * ROWS_PER_TILE, ROWS_PER_TILE)

    @pl.when(c == 0)
    def _():
        pltpu.sync_copy(acc.at[rsl], out0.at[rsl])

    @pl.when(c == 1)
    def _():
        pltpu.sync_copy(acc.at[rsl], out1.at[rsl])


_sc_spmm_call = functools.partial(
    pl.kernel,
    out_type=[
        jax.ShapeDtypeStruct((N_PAD, D_FEAT), jnp.float32),
        jax.ShapeDtypeStruct((N_PAD, D_FEAT), jnp.float32),
    ],
    mesh=plsc.VectorSubcoreMesh(core_axis_name="c", subcore_axis_name="s"),
    scratch_types=[
        pltpu.VMEM((BATCH, D_FEAT), jnp.float32),
        pltpu.VMEM((BATCH,), jnp.int32),
        pltpu.VMEM((BATCH,), jnp.int32),
        pltpu.VMEM((BATCH,), jnp.float32),
        pltpu.VMEM((ZROWS, D_FEAT), jnp.float32),
        pltpu.VMEM_SHARED((N_PAD, D_FEAT), jnp.float32),
        pltpu.SemaphoreType.DMA,
    ],
)(_sc_spmm)


def _add_body(a_ref, b_ref, o_ref):
    o_ref[...] = a_ref[...] + b_ref[...]


_combine = pl.pallas_call(
    _add_body,
    grid=(10,),
    in_specs=[
        pl.BlockSpec((N_PAD // 10, D_FEAT), lambda i: (i, 0)),
        pl.BlockSpec((N_PAD // 10, D_FEAT), lambda i: (i, 0)),
    ],
    out_specs=pl.BlockSpec((N_PAD // 10, D_FEAT), lambda i: (i, 0)),
    out_shape=jax.ShapeDtypeStruct((N_PAD, D_FEAT), jnp.float32),
)


@jax.jit
def kernel(adj_indices, adj_values, embeds):
    rows = adj_indices[0].astype(jnp.int32)
    cols = adj_indices[1].astype(jnp.int32)
    p0, p1 = _sc_spmm_call(rows, cols, adj_values, embeds)
    return _combine(p0, p1)[:N_NODES]


# speedup vs baseline: 11.6323x; 2.5839x over previous
import functools

import jax
import jax.numpy as jnp
from jax import lax
from jax.experimental import pallas as pl
from jax.experimental.pallas import tpu as pltpu
from jax.experimental.pallas import tpu_sc as plsc

N_NODES = 10000
N_EDGES = 320000
D_FEAT = 128

NC = 2
NS = 16
LANES = 16
HFEAT = D_FEAT // NC
E_PER_T = N_EDGES // NS
BATCH = 80
NBATCH = E_PER_T // BATCH
NBUF = 5
N_PAD = 10240
ROWS_PER_TILE = N_PAD // NS
NSEG = HFEAT // LANES
EPB16 = BATCH // LANES


def _sc_spmm(rows_hbm, cols_hbm, vals_hbm, embeds_hbm,
             out0, out1,
             g0, g1, g2, g3, g4, r0, r1, r2, r3, r4,
             colv, rowv, valv, acc,
             gs0, gs1, gs2, gs3, gs4, ss0, ss1, ss2, ss3, ss4):
    g = [g0, g1, g2, g3, g4]
    r = [r0, r1, r2, r3, r4]
    gsem = [gs0, gs1, gs2, gs3, gs4]
    ssem = [ss0, ss1, ss2, ss3, ss4]

    h = lax.axis_index("c")
    s = lax.axis_index("s")
    base = s * E_PER_T

    pltpu.sync_copy(cols_hbm.at[pl.ds(base, E_PER_T)], colv)
    pltpu.sync_copy(rows_hbm.at[pl.ds(base, E_PER_T)], rowv)
    pltpu.sync_copy(vals_hbm.at[pl.ds(base, E_PER_T)], valv)

    hvec = jnp.full((LANES,), h, jnp.int32)

    def xform_body(i, _):
        sl = pl.ds(i * LANES, LANES)
        colv[sl] = colv[sl] * 2 + hvec
        return 0

    lax.fori_loop(0, E_PER_T // LANES, xform_body, 0)

    def fire_gather(b, j):
        pltpu.async_copy(
            embeds_hbm.at[colv.at[pl.ds(b * BATCH, BATCH)]], g[j], gsem[j])

    def wait_gather(b, j):
        pltpu.make_async_copy(
            embeds_hbm.at[colv.at[pl.ds(b * BATCH, BATCH)]], g[j],
            gsem[j]).wait()

    def wait_scatter(j):
        pltpu.make_async_copy(g[j], acc.at[r[j]], ssem[j]).wait()

    for j in range(NBUF - 1):
        fire_gather(j, j)

    zeros16 = jnp.zeros((LANES,), jnp.float32)

    def zero_body(i, _):
        for k in range(NSEG):
            g4[i, pl.ds(k * LANES, LANES)] = zeros16
        return 0

    lax.fori_loop(0, BATCH, zero_body, 0)
    for m in range(ROWS_PER_TILE // BATCH):
        pltpu.sync_copy(g4, acc.at[pl.ds(s * ROWS_PER_TILE + m * BATCH, BATCH)])
    plsc.subcore_barrier()

    def scale(gj, b):
        def sb(j16, _):
            off = b * BATCH + j16 * LANES
            v16 = valv[pl.ds(off, LANES)]
            ebase = j16 * LANES
            for i in range(LANES):
                vv = jnp.full((LANES,), v16[i], jnp.float32)
                for k in range(NSEG):
                    sl = pl.ds(k * LANES, LANES)
                    gj[ebase + i, sl] = gj[ebase + i, sl] * vv
            return 0

        lax.fori_loop(0, EPB16, sb, 0)

    def outer(a, _):
        for j in range(NBUF):
            b = a * NBUF + j
            j4 = (j + 4) % NBUF
            wait_gather(b, j)
            scale(g[j], b)
            for i in range(EPB16):
                r[j][pl.ds(i * LANES, LANES)] = \
                    rowv[pl.ds(b * BATCH + i * LANES, LANES)]
            pltpu.async_copy(g[j], acc.at[r[j]], ssem[j], add=True)

            @pl.when(b >= 1)
            def _():
                wait_scatter(j4)

            @pl.when(b + NBUF - 1 < NBATCH)
            def _():
                fire_gather(b + NBUF - 1, j4)
        return 0

    lax.fori_loop(0, NBATCH // NBUF, outer, 0)
    wait_scatter((NBATCH - 1) % NBUF)
    plsc.subcore_barrier()

    rsl = pl.ds(s * ROWS_PER_TILE, ROWS_PER_TILE)

    @pl.when(h == 0)
    def _():
        pltpu.sync_copy(acc.at[rsl], out0.at[rsl])

    @pl.when(h == 1)
    def _():
        pltpu.sync_copy(acc.at[rsl], out1.at[rsl])


_sc_spmm_call = functools.partial(
    pl.kernel,
    out_type=[
        jax.ShapeDtypeStruct((N_PAD, HFEAT), jnp.float32),
        jax.ShapeDtypeStruct((N_PAD, HFEAT), jnp.float32),
    ],
    mesh=plsc.VectorSubcoreMesh(core_axis_name="c", subcore_axis_name="s"),
    compiler_params=pltpu.CompilerParams(use_tc_tiling_on_sc=False),
    scratch_types=(
        [pltpu.VMEM((BATCH, HFEAT), jnp.float32)] * NBUF
        + [pltpu.VMEM((BATCH,), jnp.int32)] * NBUF
        + [
            pltpu.VMEM((E_PER_T,), jnp.int32),
            pltpu.VMEM((E_PER_T,), jnp.int32),
            pltpu.VMEM((E_PER_T,), jnp.float32),
            pltpu.VMEM_SHARED((N_PAD, HFEAT), jnp.float32),
        ]
        + [pltpu.SemaphoreType.DMA] * (2 * NBUF)
    ),
)(_sc_spmm)


def _concat_body(a_ref, b_ref, o_ref):
    o_ref[:, :HFEAT] = a_ref[...]
    o_ref[:, HFEAT:] = b_ref[...]


_combine = pl.pallas_call(
    _concat_body,
    grid=(10,),
    in_specs=[
        pl.BlockSpec((N_PAD // 10, HFEAT), lambda i: (i, 0)),
        pl.BlockSpec((N_PAD // 10, HFEAT), lambda i: (i, 0)),
    ],
    out_specs=pl.BlockSpec((N_PAD // 10, D_FEAT), lambda i: (i, 0)),
    out_shape=jax.ShapeDtypeStruct((N_PAD, D_FEAT), jnp.float32),
)


@jax.jit
def kernel(adj_indices, adj_values, embeds):
    rows = adj_indices[0].astype(jnp.int32)
    cols = adj_indices[1].astype(jnp.int32)
    embeds2 = embeds.reshape(2 * N_NODES, HFEAT)
    p0, p1 = _sc_spmm_call(rows, cols, adj_values, embeds2)
    return _combine(p0, p1)[:N_NODES]


# speedup vs baseline: 12.6706x; 1.0893x over previous
import functools

import jax
import jax.numpy as jnp
from jax import lax
from jax.experimental import pallas as pl
from jax.experimental.pallas import tpu as pltpu
from jax.experimental.pallas import tpu_sc as plsc

N_NODES = 10000
N_EDGES = 320000
D_FEAT = 128

NC = 2
NS = 16
LANES = 16
HFEAT = D_FEAT // NC
E_PER_T = N_EDGES // NS
BATCH = 80
NBATCH = E_PER_T // BATCH
NBUF = 5
N_PAD = 10240
ROWS_PER_TILE = N_PAD // NS
NSEG = HFEAT // LANES
EPB16 = BATCH // LANES


def _sc_spmm(adj_hbm, vals_hbm, embeds_hbm,
             out0, out1,
             g0, g1, g2, g3, g4, r0, r1, r2, r3, r4,
             c0, c1, c2, c3, c4,
             colv, rowv, valv, acc,
             gs0, gs1, gs2, gs3, gs4, ss0, ss1, ss2, ss3, ss4):
    g = [g0, g1, g2, g3, g4]
    r = [r0, r1, r2, r3, r4]
    cb = [c0, c1, c2, c3, c4]
    gsem = [gs0, gs1, gs2, gs3, gs4]
    ssem = [ss0, ss1, ss2, ss3, ss4]

    h = lax.axis_index("c")
    s = lax.axis_index("s")
    base = s * E_PER_T

    pltpu.sync_copy(adj_hbm.at[1, pl.ds(base, E_PER_T)], colv)
    pltpu.sync_copy(adj_hbm.at[0, pl.ds(base, E_PER_T)], rowv)
    pltpu.sync_copy(vals_hbm.at[pl.ds(base, E_PER_T)], valv)

    hvec = jnp.full((LANES,), h, jnp.int32)
    two = jnp.full((LANES,), 2, jnp.int32)

    def fire_gather(b, j):
        for i in range(EPB16):
            cb[j][pl.ds(i * LANES, LANES)] = \
                colv[pl.ds(b * BATCH + i * LANES, LANES)] * two + hvec
        pltpu.async_copy(embeds_hbm.at[cb[j]], g[j], gsem[j])

    def wait_gather(b, j):
        pltpu.make_async_copy(embeds_hbm.at[cb[j]], g[j], gsem[j]).wait()

    def wait_scatter(j):
        pltpu.make_async_copy(g[j], acc.at[r[j]], ssem[j]).wait()

    for j in range(NBUF - 1):
        fire_gather(j, j)

    zeros16 = jnp.zeros((LANES,), jnp.float32)

    def zero_body(i, _):
        for k in range(NSEG):
            g4[i, pl.ds(k * LANES, LANES)] = zeros16
        return 0

    lax.fori_loop(0, BATCH, zero_body, 0)
    for m in range(ROWS_PER_TILE // BATCH):
        pltpu.sync_copy(g4, acc.at[pl.ds(s * ROWS_PER_TILE + m * BATCH, BATCH)])
    plsc.subcore_barrier()

    def scale(gj, b):
        def sb(j16, _):
            off = b * BATCH + j16 * LANES
            v16 = valv[pl.ds(off, LANES)]
            ebase = j16 * LANES
            for i in range(LANES):
                vv = jnp.full((LANES,), v16[i], jnp.float32)
                for k in range(NSEG):
                    sl = pl.ds(k * LANES, LANES)
                    gj[ebase + i, sl] = gj[ebase + i, sl] * vv
            return 0

        lax.fori_loop(0, EPB16, sb, 0)

    def outer(a, _):
        for j in range(NBUF):
            b = a * NBUF + j
            j4 = (j + 4) % NBUF
            wait_gather(b, j)
            scale(g[j], b)
            for i in range(EPB16):
                r[j][pl.ds(i * LANES, LANES)] = \
                    rowv[pl.ds(b * BATCH + i * LANES, LANES)]
            pltpu.async_copy(g[j], acc.at[r[j]], ssem[j], add=True)

            @pl.when(b >= 1)
            def _():
                wait_scatter(j4)

            @pl.when(b + NBUF - 1 < NBATCH)
            def _():
                fire_gather(b + NBUF - 1, j4)
        return 0

    lax.fori_loop(0, NBATCH // NBUF, outer, 0)
    wait_scatter((NBATCH - 1) % NBUF)
    plsc.subcore_barrier()

    rsl = pl.ds(s * ROWS_PER_TILE, ROWS_PER_TILE)

    @pl.when(h == 0)
    def _():
        pltpu.sync_copy(acc.at[rsl], out0.at[rsl])

    @pl.when(h == 1)
    def _():
        pltpu.sync_copy(acc.at[rsl], out1.at[rsl])


_sc_spmm_call = functools.partial(
    pl.kernel,
    out_type=[
        jax.ShapeDtypeStruct((N_PAD, HFEAT), jnp.float32),
        jax.ShapeDtypeStruct((N_PAD, HFEAT), jnp.float32),
    ],
    mesh=plsc.VectorSubcoreMesh(core_axis_name="c", subcore_axis_name="s"),
    compiler_params=pltpu.CompilerParams(use_tc_tiling_on_sc=False),
    scratch_types=(
        [pltpu.VMEM((BATCH, HFEAT), jnp.float32)] * NBUF
        + [pltpu.VMEM((BATCH,), jnp.int32)] * NBUF
        + [pltpu.VMEM((BATCH,), jnp.int32)] * NBUF
        + [
            pltpu.VMEM((E_PER_T,), jnp.int32),
            pltpu.VMEM((E_PER_T,), jnp.int32),
            pltpu.VMEM((E_PER_T,), jnp.float32),
            pltpu.VMEM_SHARED((N_PAD, HFEAT), jnp.float32),
        ]
        + [pltpu.SemaphoreType.DMA] * (2 * NBUF)
    ),
)(_sc_spmm)


def _concat_body(a_ref, b_ref, o_ref):
    o_ref[:, :HFEAT] = a_ref[...]
    o_ref[:, HFEAT:] = b_ref[...]


_combine = pl.pallas_call(
    _concat_body,
    grid=(10,),
    in_specs=[
        pl.BlockSpec((N_NODES // 10, HFEAT), lambda i: (i, 0)),
        pl.BlockSpec((N_NODES // 10, HFEAT), lambda i: (i, 0)),
    ],
    out_specs=pl.BlockSpec((N_NODES // 10, D_FEAT), lambda i: (i, 0)),
    out_shape=jax.ShapeDtypeStruct((N_NODES, D_FEAT), jnp.float32),
)


@jax.jit
def kernel(adj_indices, adj_values, embeds):
    adj = adj_indices.astype(jnp.int32)
    embeds2 = embeds.reshape(2 * N_NODES, HFEAT)
    p0, p1 = _sc_spmm_call(adj, adj_values, embeds2)
    return _combine(p0, p1)


# speedup vs baseline: 14.2655x; 1.1259x over previous
import functools

import jax
import jax.numpy as jnp
from jax import lax
from jax.experimental import pallas as pl
from jax.experimental.pallas import tpu as pltpu
from jax.experimental.pallas import tpu_sc as plsc

N_NODES = 10000
N_EDGES = 320000
D_FEAT = 128

NC = 2
NS = 16
LANES = 16
HFEAT = D_FEAT // NC
E_PER_T = N_EDGES // NS
BATCH = 80
NBATCH = E_PER_T // BATCH
NBUF = 5
N_PAD = 10240
ROWS_PER_TILE = N_PAD // NS
NSEG = HFEAT // LANES
EPB16 = BATCH // LANES


def _sc_spmm(adj_hbm, vals_hbm, embeds_hbm,
             out,
             g0, g1, g2, g3, g4, r0, r1, r2, r3, r4,
             c0, c1, c2, c3, c4,
             colv, rowv, valv, acc,
             gs0, gs1, gs2, gs3, gs4, ss0, ss1, ss2, ss3, ss4):
    g = [g0, g1, g2, g3, g4]
    r = [r0, r1, r2, r3, r4]
    cb = [c0, c1, c2, c3, c4]
    gsem = [gs0, gs1, gs2, gs3, gs4]
    ssem = [ss0, ss1, ss2, ss3, ss4]

    h = lax.axis_index("c")
    s = lax.axis_index("s")
    base = s * E_PER_T

    pltpu.sync_copy(adj_hbm.at[1, pl.ds(base, E_PER_T)], colv)
    pltpu.sync_copy(adj_hbm.at[0, pl.ds(base, E_PER_T)], rowv)
    pltpu.sync_copy(vals_hbm.at[pl.ds(base, E_PER_T)], valv)

    hvec = jnp.full((LANES,), h, jnp.int32)
    two = jnp.full((LANES,), 2, jnp.int32)

    def fire_gather(b, j):
        for i in range(EPB16):
            cb[j][pl.ds(i * LANES, LANES)] = \
                colv[pl.ds(b * BATCH + i * LANES, LANES)] * two + hvec
        pltpu.async_copy(embeds_hbm.at[cb[j]], g[j], gsem[j])

    def wait_gather(b, j):
        pltpu.make_async_copy(embeds_hbm.at[cb[j]], g[j], gsem[j]).wait()

    def wait_scatter(j):
        pltpu.make_async_copy(g[j], acc.at[r[j]], ssem[j]).wait()

    for j in range(NBUF - 1):
        fire_gather(j, j)

    zeros16 = jnp.zeros((LANES,), jnp.float32)

    def zero_body(i, _):
        for k in range(NSEG):
            g4[i, pl.ds(k * LANES, LANES)] = zeros16
        return 0

    lax.fori_loop(0, BATCH, zero_body, 0)
    for m in range(ROWS_PER_TILE // BATCH):
        pltpu.sync_copy(g4, acc.at[pl.ds(s * ROWS_PER_TILE + m * BATCH, BATCH)])
    plsc.subcore_barrier()

    def scale(gj, b):
        def sb(j16, _):
            off = b * BATCH + j16 * LANES
            v16 = valv[pl.ds(off, LANES)]
            ebase = j16 * LANES
            for i in range(LANES):
                vv = jnp.full((LANES,), v16[i], jnp.float32)
                for k in range(NSEG):
                    sl = pl.ds(k * LANES, LANES)
                    gj[ebase + i, sl] = gj[ebase + i, sl] * vv
            return 0

        lax.fori_loop(0, EPB16, sb, 0)

    def outer(a, _):
        for j in range(NBUF):
            b = a * NBUF + j
            j4 = (j + 4) % NBUF
            wait_gather(b, j)
            scale(g[j], b)
            for i in range(EPB16):
                r[j][pl.ds(i * LANES, LANES)] = \
                    rowv[pl.ds(b * BATCH + i * LANES, LANES)]
            pltpu.async_copy(g[j], acc.at[r[j]], ssem[j], add=True)

            @pl.when(b >= 1)
            def _():
                wait_scatter(j4)

            @pl.when(b + NBUF - 1 < NBATCH)
            def _():
                fire_gather(b + NBUF - 1, j4)
        return 0

    lax.fori_loop(0, NBATCH // NBUF, outer, 0)
    wait_scatter((NBATCH - 1) % NBUF)
    plsc.subcore_barrier()

    csl = pl.ds(h * HFEAT, HFEAT)

    @pl.when(s < NS - 1)
    def _():
        rsl = pl.ds(s * ROWS_PER_TILE, ROWS_PER_TILE)
        pltpu.sync_copy(acc.at[rsl], out.at[rsl, csl])

    @pl.when(s == NS - 1)
    def _():
        tail = N_NODES - (NS - 1) * ROWS_PER_TILE
        rslt = pl.ds((NS - 1) * ROWS_PER_TILE, tail)
        pltpu.sync_copy(acc.at[rslt], out.at[rslt, csl])


_sc_spmm_call = functools.partial(
    pl.kernel,
    out_type=jax.ShapeDtypeStruct((N_NODES, D_FEAT), jnp.float32),
    mesh=plsc.VectorSubcoreMesh(core_axis_name="c", subcore_axis_name="s"),
    compiler_params=pltpu.CompilerParams(use_tc_tiling_on_sc=False),
    scratch_types=(
        [pltpu.VMEM((BATCH, HFEAT), jnp.float32)] * NBUF
        + [pltpu.VMEM((BATCH,), jnp.int32)] * NBUF
        + [pltpu.VMEM((BATCH,), jnp.int32)] * NBUF
        + [
            pltpu.VMEM((E_PER_T,), jnp.int32),
            pltpu.VMEM((E_PER_T,), jnp.int32),
            pltpu.VMEM((E_PER_T,), jnp.float32),
            pltpu.VMEM_SHARED((N_PAD, HFEAT), jnp.float32),
        ]
        + [pltpu.SemaphoreType.DMA] * (2 * NBUF)
    ),
)(_sc_spmm)


@jax.jit
def kernel(adj_indices, adj_values, embeds):
    adj = adj_indices.astype(jnp.int32)
    embeds2 = embeds.reshape(2 * N_NODES, HFEAT)
    return _sc_spmm_call(adj, adj_values, embeds2)


# speedup vs baseline: 14.4099x; 1.0101x over previous
import functools

import jax
import jax.numpy as jnp
from jax import lax
from jax.experimental import pallas as pl
from jax.experimental.pallas import tpu as pltpu
from jax.experimental.pallas import tpu_sc as plsc

N_NODES = 10000
N_EDGES = 320000
D_FEAT = 128

NC = 2
NS = 16
LANES = 16
HFEAT = D_FEAT // NC
E_PER_T = N_EDGES // NS
BATCH = 80
NBATCH = E_PER_T // BATCH
NBUF = 5
N_PAD = 10240
ROWS_PER_TILE = N_PAD // NS
NSEG = HFEAT // LANES
EPB16 = BATCH // LANES


def _sc_spmm(adj_hbm, vals_hbm, embeds_hbm,
             out,
             g0, g1, g2, g3, g4, r0, r1, r2, r3, r4,
             c0, c1, c2, c3, c4,
             colv, rowv, valv, acc,
             gs0, gs1, gs2, gs3, gs4, ss0, ss1, ss2, ss3, ss4,
             pc, pr, pv):
    g = [g0, g1, g2, g3, g4]
    r = [r0, r1, r2, r3, r4]
    cb = [c0, c1, c2, c3, c4]
    gsem = [gs0, gs1, gs2, gs3, gs4]
    ssem = [ss0, ss1, ss2, ss3, ss4]

    h = lax.axis_index("c")
    s = lax.axis_index("s")
    base = s * E_PER_T

    pltpu.async_copy(adj_hbm.at[1, pl.ds(base, E_PER_T)], colv, pc)
    pltpu.async_copy(adj_hbm.at[0, pl.ds(base, E_PER_T)], rowv, pr)
    pltpu.async_copy(vals_hbm.at[pl.ds(base, E_PER_T)], valv, pv)

    hvec = jnp.full((LANES,), h, jnp.int32)
    two = jnp.full((LANES,), 2, jnp.int32)

    def fire_gather(b, j):
        for i in range(EPB16):
            cb[j][pl.ds(i * LANES, LANES)] = \
                colv[pl.ds(b * BATCH + i * LANES, LANES)] * two + hvec
        pltpu.async_copy(embeds_hbm.at[cb[j]], g[j], gsem[j])

    def wait_gather(b, j):
        pltpu.make_async_copy(embeds_hbm.at[cb[j]], g[j], gsem[j]).wait()

    def wait_scatter(j):
        pltpu.make_async_copy(g[j], acc.at[r[j]], ssem[j]).wait()

    pltpu.make_async_copy(adj_hbm.at[1, pl.ds(base, E_PER_T)], colv, pc).wait()
    for j in range(NBUF - 1):
        fire_gather(j, j)

    zeros16 = jnp.zeros((LANES,), jnp.float32)

    def zero_body(i, _):
        for k in range(NSEG):
            g4[i, pl.ds(k * LANES, LANES)] = zeros16
        return 0

    lax.fori_loop(0, BATCH, zero_body, 0)
    for m in range(ROWS_PER_TILE // BATCH):
        pltpu.sync_copy(g4, acc.at[pl.ds(s * ROWS_PER_TILE + m * BATCH, BATCH)])
    pltpu.make_async_copy(adj_hbm.at[0, pl.ds(base, E_PER_T)], rowv, pr).wait()
    pltpu.make_async_copy(vals_hbm.at[pl.ds(base, E_PER_T)], valv, pv).wait()
    plsc.subcore_barrier()

    def scale(gj, b):
        def sb(j16, _):
            off = b * BATCH + j16 * LANES
            v16 = valv[pl.ds(off, LANES)]
            ebase = j16 * LANES
            for i in range(LANES):
                vv = jnp.full((LANES,), v16[i], jnp.float32)
                for k in range(NSEG):
                    sl = pl.ds(k * LANES, LANES)
                    gj[ebase + i, sl] = gj[ebase + i, sl] * vv
            return 0

        lax.fori_loop(0, EPB16, sb, 0)

    def outer(a, _):
        for j in range(NBUF):
            b = a * NBUF + j
            j4 = (j + 4) % NBUF
            wait_gather(b, j)
            scale(g[j], b)
            for i in range(EPB16):
                r[j][pl.ds(i * LANES, LANES)] = \
                    rowv[pl.ds(b * BATCH + i * LANES, LANES)]
            pltpu.async_copy(g[j], acc.at[r[j]], ssem[j], add=True)

            @pl.when(b >= 1)
            def _():
                wait_scatter(j4)

            @pl.when(b + NBUF - 1 < NBATCH)
            def _():
                fire_gather(b + NBUF - 1, j4)
        return 0

    lax.fori_loop(0, NBATCH // NBUF, outer, 0)
    wait_scatter((NBATCH - 1) % NBUF)
    plsc.subcore_barrier()

    csl = pl.ds(h * HFEAT, HFEAT)

    @pl.when(s < NS - 1)
    def _():
        rsl = pl.ds(s * ROWS_PER_TILE, ROWS_PER_TILE)
        pltpu.sync_copy(acc.at[rsl], out.at[rsl, csl])

    @pl.when(s == NS - 1)
    def _():
        tail = N_NODES - (NS - 1) * ROWS_PER_TILE
        rslt = pl.ds((NS - 1) * ROWS_PER_TILE, tail)
        pltpu.sync_copy(acc.at[rslt], out.at[rslt, csl])


_sc_spmm_call = functools.partial(
    pl.kernel,
    out_type=jax.ShapeDtypeStruct((N_NODES, D_FEAT), jnp.float32),
    mesh=plsc.VectorSubcoreMesh(core_axis_name="c", subcore_axis_name="s"),
    compiler_params=pltpu.CompilerParams(use_tc_tiling_on_sc=False),
    scratch_types=(
        [pltpu.VMEM((BATCH, HFEAT), jnp.float32)] * NBUF
        + [pltpu.VMEM((BATCH,), jnp.int32)] * NBUF
        + [pltpu.VMEM((BATCH,), jnp.int32)] * NBUF
        + [
            pltpu.VMEM((E_PER_T,), jnp.int32),
            pltpu.VMEM((E_PER_T,), jnp.int32),
            pltpu.VMEM((E_PER_T,), jnp.float32),
            pltpu.VMEM_SHARED((N_PAD, HFEAT), jnp.float32),
        ]
        + [pltpu.SemaphoreType.DMA] * (2 * NBUF + 3)
    ),
)(_sc_spmm)


@jax.jit
def kernel(adj_indices, adj_values, embeds):
    adj = adj_indices.astype(jnp.int32)
    embeds2 = embeds.reshape(2 * N_NODES, HFEAT)
    return _sc_spmm_call(adj, adj_values, embeds2)
